# baseline (speedup 1.0000x reference)
import jax
import jax.numpy as jnp
from jax.experimental import pallas as pl
from jax.experimental.pallas import tpu as pltpu

N_WCHUNK = 4
S_BLK = 512


def _mm_kernel(x_hbm, w_hbm, out_hbm, wv_ref, xb_ref, ob_ref,
               wsem, xsem, osem):
    B, S, I = x_hbm.shape
    O = w_hbm.shape[0]
    WC = O // N_WCHUNK
    n_s = S // S_BLK
    n_q = B * n_s

    def x_copy(q):
        b, s = divmod(q, n_s)
        return pltpu.make_async_copy(
            x_hbm.at[b, pl.ds(s * S_BLK, S_BLK), :], xb_ref.at[q % 2],
            xsem.at[q % 2])

    def out_copy(q):
        b, s = divmod(q, n_s)
        return pltpu.make_async_copy(
            ob_ref.at[q % 2], out_hbm.at[b, :, pl.ds(s * S_BLK, S_BLK)],
            osem.at[q % 2])

    for wq in range(N_WCHUNK):
        pltpu.make_async_copy(
            w_hbm.at[pl.ds(wq * WC, WC), :], wv_ref.at[pl.ds(wq * WC, WC), :],
            wsem.at[wq]).start()
    x_copy(0).start()

    for q in range(n_q):
        if q + 1 < n_q:
            x_copy(q + 1).start()
        x_copy(q).wait()
        if q >= 2:
            out_copy(q - 2).wait()
        if q == 0:
            for wq in range(N_WCHUNK):
                pltpu.make_async_copy(
                    w_hbm.at[pl.ds(wq * WC, WC), :],
                    wv_ref.at[pl.ds(wq * WC, WC), :], wsem.at[wq]).wait()
                ob_ref[0, pl.ds(wq * WC, WC), :] = jax.lax.dot_general(
                    wv_ref[pl.ds(wq * WC, WC), :], xb_ref[0],
                    (((1,), (1,)), ((), ())),
                    preferred_element_type=jnp.float32)
        else:
            ob_ref[q % 2] = jax.lax.dot_general(
                wv_ref[...], xb_ref[q % 2],
                (((1,), (1,)), ((), ())),
                preferred_element_type=jnp.float32)
        out_copy(q).start()

    out_copy(n_q - 2).wait()
    out_copy(n_q - 1).wait()


@jax.jit
def kernel(x, weight):
    B, S, I = x.shape
    O = weight.shape[0]
    return pl.pallas_call(
        _mm_kernel,
        in_specs=[
            pl.BlockSpec(memory_space=pl.ANY),
            pl.BlockSpec(memory_space=pl.ANY),
        ],
        out_specs=pl.BlockSpec(memory_space=pl.ANY),
        out_shape=jax.ShapeDtypeStruct((B, O, S), jnp.float32),
        scratch_shapes=[
            pltpu.VMEM((O, I), jnp.float32),
            pltpu.VMEM((2, S_BLK, I), jnp.float32),
            pltpu.VMEM((2, O, S_BLK), jnp.float32),
            pltpu.SemaphoreType.DMA((N_WCHUNK,)),
            pltpu.SemaphoreType.DMA((2,)),
            pltpu.SemaphoreType.DMA((2,)),
        ],
    )(x, weight)


# speedup vs baseline: 1.0406x; 1.0406x over previous
import jax
import jax.numpy as jnp
from jax.experimental import pallas as pl
from jax.experimental.pallas import tpu as pltpu

N_CHUNK = 4


def _mm_kernel(x_ref, w_hbm, out_ref, wv_ref, sems):
    first = (pl.program_id(0) == 0) & (pl.program_id(1) == 0)
    O = wv_ref.shape[0]
    C = O // N_CHUNK

    @pl.when(first)
    def _first_step():
        for q in range(N_CHUNK):
            pltpu.make_async_copy(
                w_hbm.at[pl.ds(q * C, C), :], wv_ref.at[pl.ds(q * C, C), :],
                sems.at[q]).start()
        for q in range(N_CHUNK):
            pltpu.make_async_copy(
                w_hbm.at[pl.ds(q * C, C), :], wv_ref.at[pl.ds(q * C, C), :],
                sems.at[q]).wait()
            out_ref[0, pl.ds(q * C, C), :] = jax.lax.dot_general(
                wv_ref[pl.ds(q * C, C), :], x_ref[0],
                (((1,), (1,)), ((), ())), preferred_element_type=jnp.float32)

    @pl.when(jnp.logical_not(first))
    def _rest():
        out_ref[0] = jax.lax.dot_general(
            wv_ref[...], x_ref[0],
            (((1,), (1,)), ((), ())), preferred_element_type=jnp.float32)


@jax.jit
def kernel(x, weight):
    B, S, I = x.shape
    O = weight.shape[0]
    S_BLK = min(S, 512)

    grid = (B, S // S_BLK)
    return pl.pallas_call(
        _mm_kernel,
        grid=grid,
        in_specs=[
            pl.BlockSpec((1, S_BLK, I), lambda b, s: (b, s, 0)),
            pl.BlockSpec(memory_space=pl.ANY),
        ],
        out_specs=pl.BlockSpec((1, O, S_BLK), lambda b, s: (b, 0, s)),
        out_shape=jax.ShapeDtypeStruct((B, O, S), jnp.float32),
        scratch_shapes=[
            pltpu.VMEM((O, I), jnp.float32),
            pltpu.SemaphoreType.DMA((N_CHUNK,)),
        ],
        compiler_params=pltpu.CompilerParams(
            dimension_semantics=("arbitrary", "arbitrary"),
        ),
    )(x, weight)
